# Initial kernel scaffold; baseline (speedup 1.0000x reference)
import functools

import numpy as np
import jax
import jax.numpy as jnp
from jax import lax
from jax.experimental import pallas as pl
from jax.experimental.pallas import tpu as pltpu
from jax.experimental.pallas import tpu_sc as plsc

F32 = jnp.float32
I32 = jnp.int32

B = 2
N = 20000
NP = 20480
M = 64
NW = 16
PW = NP // NW
CH = PW // 16

D2R = np.float32(np.pi / 180.0)
BETA = np.float32(1.0 / 9.0)
HOB = np.float32(0.5 / np.float32(1.0 / 9.0))
HB = np.float32(0.5 * np.float32(1.0 / 9.0))
LN2 = np.float32(0.6931471805599453)
NEG = np.float32(-1e9)
FMIN = np.float32(-3.4e38)


def _sin(x):
    x2 = x * x
    return x * (1.0 + x2 * (np.float32(-1 / 6) + x2 * (np.float32(1 / 120)
            + x2 * (np.float32(-1 / 5040) + x2 * np.float32(1 / 362880)))))


def _cos(x):
    x2 = x * x
    return 1.0 + x2 * (np.float32(-0.5) + x2 * (np.float32(1 / 24)
            + x2 * (np.float32(-1 / 720) + x2 * np.float32(1 / 40320))))


def _log(w):
    xi = plsc.bitcast(w, I32)
    e = lax.shift_right_logical(xi, 23) - 127
    m = plsc.bitcast(jnp.bitwise_or(jnp.bitwise_and(xi, 0x7FFFFF), 0x3F800000), F32)
    big = m > np.float32(1.4142135)
    m = jnp.where(big, m * np.float32(0.5), m)
    e = e + big.astype(I32)
    z = (m - 1.0) / (m + 1.0)
    z2 = z * z
    lm = 2.0 * z * (1.0 + z2 * (np.float32(1 / 3) + z2 * (np.float32(1 / 5)
            + z2 * (np.float32(1 / 7) + z2 * np.float32(1 / 9)))))
    return e.astype(F32) * LN2 + lm


def _make_body():
    mesh = plsc.VectorSubcoreMesh(core_axis_name="c", subcore_axis_name="s")
    scratch = [
        pltpu.VMEM((PW * 5,), F32),
        pltpu.VMEM((PW * 5,), F32),
        pltpu.VMEM((M * 6,), F32),
        pltpu.VMEM((PW,), F32),
        pltpu.VMEM((PW,), F32),
        pltpu.VMEM((PW,), F32),
        pltpu.VMEM((PW,), F32),
        pltpu.VMEM((PW,), F32),
        pltpu.VMEM((PW,), F32),
        pltpu.VMEM((PW,), F32),
        pltpu.VMEM((PW,), F32),
        pltpu.VMEM((PW,), F32),
        pltpu.VMEM((PW,), F32),
        pltpu.VMEM((PW,), F32),
        pltpu.VMEM((PW,), F32),
        pltpu.VMEM((PW,), F32),
        pltpu.VMEM((PW,), F32),
        pltpu.VMEM((PW,), F32),
        pltpu.VMEM((PW,), F32),
        pltpu.VMEM((PW,), I32),
        pltpu.VMEM((PW,), F32),
        pltpu.VMEM((M,), F32),
        pltpu.VMEM((M,), F32),
        pltpu.VMEM((M,), F32),
        pltpu.VMEM((M,), F32),
        pltpu.VMEM((M,), F32),
        pltpu.VMEM((M,), F32),
        pltpu.VMEM((M,), F32),
        pltpu.VMEM((M,), F32),
        pltpu.VMEM((M,), F32),
        pltpu.VMEM((M,), F32),
        pltpu.VMEM((M,), F32),
        pltpu.VMEM((M,), F32),
        pltpu.VMEM((M,), F32),
        pltpu.VMEM((M,), F32),
        pltpu.VMEM((M,), F32),
        pltpu.VMEM((M,), F32),
        pltpu.VMEM((M,), F32),
        pltpu.VMEM((M,), I32),
        pltpu.VMEM((NW * M,), F32),
        pltpu.VMEM((NW * M,), I32),
        pltpu.VMEM((NW * 16,), F32),
        pltpu.VMEM((16,), F32),
        pltpu.VMEM_SHARED((NW * M,), F32),
        pltpu.VMEM_SHARED((NW * M,), I32),
        pltpu.VMEM_SHARED((NW * 16,), F32),
    ]

    @functools.partial(
        pl.kernel, mesh=mesh,
        out_type=jax.ShapeDtypeStruct((B, 16), F32),
        scratch_types=scratch,
    )
    def body(reg_hbm, anc_hbm, ann_hbm, out_hbm,
             a_aos, r_aos, ann_aos,
             asx1, asy1, asx2, asy2, asA, aax1, aay1, aax2, aay2, aaA,
             aew, aeh, alw, alh, atn,
             amax, aarg, extra,
             gsx1, gsy1, gsx2, gsy2, gsA, gax1, gay1, gax2, gay2, gaA,
             gvalid, ggx, ggy, glw, glh, gtn,
             wgmax, wgarg, cgmax, cgarg, cacc, st16,
             sh_gmax, sh_garg, sh_acc):
        j = lax.axis_index("c")
        s = lax.axis_index("s")
        iota = lax.iota(I32, 16)
        zf = jnp.zeros((16,), F32)
        base = s * PW

        pltpu.sync_copy(anc_hbm.at[pl.ds(j * (NP * 5) + base * 5, PW * 5)], a_aos)
        pltpu.sync_copy(reg_hbm.at[pl.ds(j * (NP * 5) + base * 5, PW * 5)], r_aos)
        pltpu.sync_copy(ann_hbm.at[pl.ds(j * (M * 6), M * 6)], ann_aos)

        nvalid = zf
        for g in range(M // 16):
            rows = g * 16 + iota
            gx = plsc.load_gather(ann_aos, [rows * 6])
            gy = plsc.load_gather(ann_aos, [rows * 6 + 1])
            gw = plsc.load_gather(ann_aos, [rows * 6 + 2])
            gh = plsc.load_gather(ann_aos, [rows * 6 + 3])
            gt = plsc.load_gather(ann_aos, [rows * 6 + 4])
            gl = plsc.load_gather(ann_aos, [rows * 6 + 5])
            sl = pl.ds(g * 16, 16)
            vf = jnp.where(gl != np.float32(-1.0), jnp.ones((16,), F32), zf)
            gvalid[sl] = vf
            nvalid = nvalid + vf
            gs = jnp.maximum(gw, gh)
            h = gs * np.float32(0.5)
            x1 = gx - h; y1 = gy - h; x2 = gx + h; y2 = gy + h
            gsx1[sl] = x1; gsy1[sl] = y1; gsx2[sl] = x2; gsy2[sl] = y2
            gsA[sl] = (x2 - x1) * (y2 - y1)
            rad = gt * D2R
            cs = _cos(rad); sn = _sin(rad)
            ca = jnp.abs(cs); sa = jnp.abs(sn)
            ew = (gw * ca + gh * sa) * np.float32(0.5)
            eh = (gw * sa + gh * ca) * np.float32(0.5)
            bx1 = gx - ew; by1 = gy - eh; bx2 = gx + ew; by2 = gy + eh
            gax1[sl] = bx1; gay1[sl] = by1; gax2[sl] = bx2; gay2[sl] = by2
            gaA[sl] = (bx2 - bx1) * (by2 - by1)
            ggx[sl] = gx; ggy[sl] = gy
            glw[sl] = _log(jnp.maximum(gw, 1.0))
            glh[sl] = _log(jnp.maximum(gh, 1.0))
            gtn[sl] = sn / cs
        nvalid_s = jnp.sum(nvalid)

        def pre_a(c, _):
            rows = c * 16 + iota
            sl = pl.ds(c * 16, 16)
            ax = plsc.load_gather(a_aos, [rows * 5])
            ay = plsc.load_gather(a_aos, [rows * 5 + 1])
            aw = plsc.load_gather(a_aos, [rows * 5 + 2])
            ah = plsc.load_gather(a_aos, [rows * 5 + 3])
            at = plsc.load_gather(a_aos, [rows * 5 + 4])
            ss = jnp.maximum(aw, ah)
            h = ss * np.float32(0.5)
            x1 = ax - h; y1 = ay - h; x2 = ax + h; y2 = ay + h
            asx1[sl] = x1; asy1[sl] = y1; asx2[sl] = x2; asy2[sl] = y2
            asA[sl] = (x2 - x1) * (y2 - y1)
            rad = at * D2R
            cs = _cos(rad); sn = _sin(rad)
            ca = jnp.abs(cs); sa = jnp.abs(sn)
            ew = (aw * ca + ah * sa) * np.float32(0.5)
            eh = (aw * sa + ah * ca) * np.float32(0.5)
            bx1 = ax - ew; by1 = ay - eh; bx2 = ax + ew; by2 = ay + eh
            aax1[sl] = bx1; aay1[sl] = by1; aax2[sl] = bx2; aay2[sl] = by2
            aaA[sl] = (bx2 - bx1) * (by2 - by1)
            ew2 = jnp.maximum(aw, 1.0); eh2 = jnp.maximum(ah, 1.0)
            aew[sl] = ew2; aeh[sl] = eh2
            alw[sl] = _log(ew2); alh[sl] = _log(eh2)
            atn[sl] = sn / cs
            amax[sl] = jnp.full((16,), FMIN, F32)
            aarg[sl] = jnp.zeros((16,), I32)
            extra[sl] = zf
            return 0
        lax.fori_loop(0, CH, pre_a, 0)

        def m_body(m, _):
            mi = jnp.full((16,), m, I32)
            bsx1 = plsc.load_gather(gsx1, [mi])
            bsy1 = plsc.load_gather(gsy1, [mi])
            bsx2 = plsc.load_gather(gsx2, [mi])
            bsy2 = plsc.load_gather(gsy2, [mi])
            bsA = plsc.load_gather(gsA, [mi])
            bax1 = plsc.load_gather(gax1, [mi])
            bay1 = plsc.load_gather(gay1, [mi])
            bax2 = plsc.load_gather(gax2, [mi])
            bay2 = plsc.load_gather(gay2, [mi])
            baA = plsc.load_gather(gaA, [mi])
            bval = plsc.load_gather(gvalid, [mi]) > np.float32(0.5)

            def c_body(c, carry):
                gmx, grc = carry
                sl = pl.ds(c * 16, 16)
                ix1 = jnp.maximum(asx1[sl], bsx1)
                iy1 = jnp.maximum(asy1[sl], bsy1)
                ix2 = jnp.minimum(asx2[sl], bsx2)
                iy2 = jnp.minimum(asy2[sl], bsy2)
                iw = jnp.maximum(ix2 - ix1, 0.0)
                ih = jnp.maximum(iy2 - iy1, 0.0)
                inter = iw * ih
                union = asA[sl] + bsA - inter
                ind_ok = inter >= np.float32(0.1) * union
                jx1 = jnp.maximum(aax1[sl], bax1)
                jy1 = jnp.maximum(aay1[sl], bay1)
                jx2 = jnp.minimum(aax2[sl], bax2)
                jy2 = jnp.minimum(aay2[sl], bay2)
                jw = jnp.maximum(jx2 - jx1, 0.0)
                jh = jnp.maximum(jy2 - jy1, 0.0)
                inter2 = jw * jh
                union2 = aaA[sl] + baA - inter2
                iou = inter2 / union2
                ovv = jnp.where(ind_ok, iou, 0.0)
                ovv = jnp.where(bval, ovv, jnp.full((16,), NEG, F32))
                am = amax[sl]
                take = ovv > am
                amax[sl] = jnp.where(take, ovv, am)
                aarg[sl] = jnp.where(take, mi, aarg[sl])
                t2 = ovv > gmx
                gmx = jnp.where(t2, ovv, gmx)
                grc = jnp.where(t2, jnp.full((16,), c, I32), grc)
                return gmx, grc

            gmx, grc = lax.fori_loop(
                0, CH, c_body,
                (jnp.full((16,), FMIN, F32), jnp.zeros((16,), I32)))
            topv = jnp.max(gmx)
            cand = jnp.where(gmx == topv, grc * 16 + iota + base,
                             jnp.full((16,), 2147483647, I32))
            argi = jnp.min(cand)
            lane0 = iota == 0
            plsc.store_scatter(wgmax, [mi], jnp.full((16,), topv, F32), mask=lane0)
            plsc.store_scatter(wgarg, [mi], jnp.full((16,), argi, I32), mask=lane0)
            return 0
        lax.fori_loop(0, M, m_body, 0)

        pltpu.sync_copy(wgmax, sh_gmax.at[pl.ds(s * M, M)])
        pltpu.sync_copy(wgarg, sh_garg.at[pl.ds(s * M, M)])
        plsc.subcore_barrier()
        pltpu.sync_copy(sh_gmax, cgmax)
        pltpu.sync_copy(sh_garg, cgarg)
        ones = jnp.ones((16,), F32)
        for g in range(M // 16):
            bm = cgmax[pl.ds(g * 16, 16)]
            ba = cgarg[pl.ds(g * 16, 16)]
            for w in range(1, NW):
                wm = cgmax[pl.ds(w * M + g * 16, 16)]
                wa = cgarg[pl.ds(w * M + g * 16, 16)]
                t = wm > bm
                ba = jnp.where(t, wa, ba)
                bm = jnp.where(t, wm, bm)
            gv = gvalid[pl.ds(g * 16, 16)] > np.float32(0.5)
            force = (bm < np.float32(0.5)) & gv
            idl = ba - base
            msk = force & (idl >= 0) & (idl < PW)
            idl = jnp.clip(idl, 0, PW - 1)
            plsc.store_scatter(extra, [idl], ones, mask=msk)

        def p2_body(c, carry):
            accL, accP = carry
            sl = pl.ds(c * 16, 16)
            rows = c * 16 + iota
            am = amax[sl]
            ag = aarg[sl]
            posb = (am >= np.float32(0.5)) | (extra[sl] > np.float32(0.5))
            gxv = plsc.load_gather(ggx, [ag])
            gyv = plsc.load_gather(ggy, [ag])
            glwv = plsc.load_gather(glw, [ag])
            glhv = plsc.load_gather(glh, [ag])
            gtnv = plsc.load_gather(gtn, [ag])
            axv = plsc.load_gather(a_aos, [rows * 5])
            ayv = plsc.load_gather(a_aos, [rows * 5 + 1])
            tdx = np.float32(10.0) * (gxv - axv) / aew[sl]
            tdy = np.float32(10.0) * (gyv - ayv) / aeh[sl]
            tdw = np.float32(10.0) * (glwv - alw[sl])
            tdh = np.float32(10.0) * (glhv - alh[sl])
            tdt = np.float32(15.0) * (gtnv - atn[sl])
            ssum = zf
            for k, td in enumerate((tdx, tdy, tdw, tdh, tdt)):
                rv = plsc.load_gather(r_aos, [rows * 5 + k])
                d = jnp.abs(rv - td)
                e = jnp.where(d < BETA, HOB * d * d, d - HB)
                ssum = ssum + e
            accL = accL + jnp.where(posb, ssum, zf)
            accP = accP + jnp.where(posb, jnp.ones((16,), F32), zf)
            return accL, accP
        accL, accP = lax.fori_loop(0, CH, p2_body, (zf, zf))

        lsum = jnp.sum(accL)
        psum = jnp.sum(accP)
        row = jnp.where(iota == 0, jnp.full((16,), lsum, F32),
                        jnp.where(iota == 1, jnp.full((16,), psum, F32), zf))
        st16[pl.ds(0, 16)] = row
        pltpu.sync_copy(st16, sh_acc.at[pl.ds(s * 16, 16)])
        plsc.subcore_barrier()

        @pl.when(s == 0)
        def _():
            pltpu.sync_copy(sh_acc, cacc)
            tot = zf
            for w in range(NW):
                tot = tot + cacc[pl.ds(w * 16, 16)]
            st16[pl.ds(0, 16)] = tot
            lv = plsc.load_gather(st16, [jnp.zeros((16,), I32)])
            pv = plsc.load_gather(st16, [jnp.ones((16,), I32)])
            denom = jnp.maximum(pv * np.float32(5.0), 1.0)
            res = lv / denom
            ok = (pv > 0.0) & (jnp.full((16,), nvalid_s, F32) > 0.0)
            outrow = jnp.where(ok & (iota == 0), res, zf)
            st16[pl.ds(0, 16)] = outrow
            pltpu.sync_copy(st16, out_hbm.at[j])

    return body


_body = _make_body()


@jax.jit
def kernel(regressions, anchors, annotations):
    pad_a = jnp.zeros((B, NP - N, 5), F32)
    pad_a = pad_a.at[:, :, 0].set(-1e6).at[:, :, 1].set(-1e6)
    pad_a = pad_a.at[:, :, 2].set(16.0).at[:, :, 3].set(16.0)
    anc = jnp.concatenate([anchors, pad_a], axis=1).reshape(-1)
    reg = jnp.concatenate(
        [regressions, jnp.zeros((B, NP - N, 5), F32)], axis=1).reshape(-1)
    ann = annotations.reshape(-1)
    out = _body(reg, anc, ann)
    return jnp.mean(out[:, 0], keepdims=True)


# speedup vs baseline: 2.3843x; 2.3843x over previous
import functools

import numpy as np
import jax
import jax.numpy as jnp
from jax import lax
from jax.experimental import pallas as pl
from jax.experimental.pallas import tpu as pltpu
from jax.experimental.pallas import tpu_sc as plsc

F32 = jnp.float32
I32 = jnp.int32

B = 2
N = 20000
NP = 20480
M = 64
NW = 16
PW = NP // NW
CH = PW // 16

D2R = np.float32(np.pi / 180.0)
BETA = np.float32(1.0 / 9.0)
HOB = np.float32(0.5 / np.float32(1.0 / 9.0))
HB = np.float32(0.5 * np.float32(1.0 / 9.0))
LN2 = np.float32(0.6931471805599453)
NEG = np.float32(-1e9)
FMIN = np.float32(-3.4e38)


def _sin(x):
    x2 = x * x
    return x * (1.0 + x2 * (np.float32(-1 / 6) + x2 * (np.float32(1 / 120)
            + x2 * (np.float32(-1 / 5040) + x2 * np.float32(1 / 362880)))))


def _cos(x):
    x2 = x * x
    return 1.0 + x2 * (np.float32(-0.5) + x2 * (np.float32(1 / 24)
            + x2 * (np.float32(-1 / 720) + x2 * np.float32(1 / 40320))))


def _log(w):
    xi = plsc.bitcast(w, I32)
    e = lax.shift_right_logical(xi, 23) - 127
    m = plsc.bitcast(jnp.bitwise_or(jnp.bitwise_and(xi, 0x7FFFFF), 0x3F800000), F32)
    big = m > np.float32(1.4142135)
    m = jnp.where(big, m * np.float32(0.5), m)
    e = e + big.astype(I32)
    z = (m - 1.0) / (m + 1.0)
    z2 = z * z
    lm = 2.0 * z * (1.0 + z2 * (np.float32(1 / 3) + z2 * (np.float32(1 / 5)
            + z2 * (np.float32(1 / 7) + z2 * np.float32(1 / 9)))))
    return e.astype(F32) * LN2 + lm


def _make_body():
    mesh = plsc.VectorSubcoreMesh(core_axis_name="c", subcore_axis_name="s")
    scratch = [
        pltpu.VMEM((PW * 5,), F32),
        pltpu.VMEM((PW * 5,), F32),
        pltpu.VMEM((M * 6,), F32),
        pltpu.VMEM((PW,), F32),
        pltpu.VMEM((PW,), F32),
        pltpu.VMEM((PW,), F32),
        pltpu.VMEM((PW,), F32),
        pltpu.VMEM((PW,), F32),
        pltpu.VMEM((PW,), F32),
        pltpu.VMEM((PW,), F32),
        pltpu.VMEM((PW,), F32),
        pltpu.VMEM((PW,), F32),
        pltpu.VMEM((PW,), F32),
        pltpu.VMEM((PW,), F32),
        pltpu.VMEM((PW,), F32),
        pltpu.VMEM((PW,), F32),
        pltpu.VMEM((PW,), F32),
        pltpu.VMEM((PW,), F32),
        pltpu.VMEM((PW,), F32),
        pltpu.VMEM((PW,), I32),
        pltpu.VMEM((PW,), F32),
        pltpu.VMEM((M,), F32),
        pltpu.VMEM((M,), F32),
        pltpu.VMEM((M,), F32),
        pltpu.VMEM((M,), F32),
        pltpu.VMEM((M,), F32),
        pltpu.VMEM((M,), F32),
        pltpu.VMEM((M,), F32),
        pltpu.VMEM((M,), F32),
        pltpu.VMEM((M,), F32),
        pltpu.VMEM((M,), F32),
        pltpu.VMEM((M,), F32),
        pltpu.VMEM((M,), F32),
        pltpu.VMEM((M,), F32),
        pltpu.VMEM((M,), F32),
        pltpu.VMEM((M,), F32),
        pltpu.VMEM((M,), F32),
        pltpu.VMEM((M,), F32),
        pltpu.VMEM((M,), I32),
        pltpu.VMEM((NW * M,), F32),
        pltpu.VMEM((NW * M,), I32),
        pltpu.VMEM((NW * 16,), F32),
        pltpu.VMEM((16,), F32),
        pltpu.VMEM_SHARED((NW * M,), F32),
        pltpu.VMEM_SHARED((NW * M,), I32),
        pltpu.VMEM_SHARED((NW * 16,), F32),
    ]

    @functools.partial(
        pl.kernel, mesh=mesh,
        out_type=jax.ShapeDtypeStruct((B, 16), F32),
        scratch_types=scratch,
        compiler_params=pltpu.CompilerParams(needs_layout_passes=False),
    )
    def body(reg_hbm, anc_hbm, ann_hbm, out_hbm,
             a_aos, r_aos, ann_aos,
             asx1, asy1, asx2, asy2, asA, aax1, aay1, aax2, aay2, aaA,
             aew, aeh, alw, alh, atn,
             amax, aarg, extra,
             gsx1, gsy1, gsx2, gsy2, gsA, gax1, gay1, gax2, gay2, gaA,
             gvalid, ggx, ggy, glw, glh, gtn,
             wgmax, wgarg, cgmax, cgarg, cacc, st16,
             sh_gmax, sh_garg, sh_acc):
        j = lax.axis_index("c")
        s = lax.axis_index("s")
        iota = lax.iota(I32, 16)
        zf = jnp.zeros((16,), F32)
        base = s * PW

        pltpu.sync_copy(anc_hbm.at[pl.ds(j * (NP * 5) + base * 5, PW * 5)], a_aos)
        pltpu.sync_copy(reg_hbm.at[pl.ds(j * (NP * 5) + base * 5, PW * 5)], r_aos)
        pltpu.sync_copy(ann_hbm.at[pl.ds(j * (M * 6), M * 6)], ann_aos)

        nvalid = zf
        for g in range(M // 16):
            rows = g * 16 + iota
            gx = plsc.load_gather(ann_aos, [rows * 6])
            gy = plsc.load_gather(ann_aos, [rows * 6 + 1])
            gw = plsc.load_gather(ann_aos, [rows * 6 + 2])
            gh = plsc.load_gather(ann_aos, [rows * 6 + 3])
            gt = plsc.load_gather(ann_aos, [rows * 6 + 4])
            gl = plsc.load_gather(ann_aos, [rows * 6 + 5])
            sl = pl.ds(g * 16, 16)
            vf = jnp.where(gl != np.float32(-1.0), jnp.ones((16,), F32), zf)
            gvalid[sl] = vf
            nvalid = nvalid + vf
            gs = jnp.maximum(gw, gh)
            h = gs * np.float32(0.5)
            x1 = gx - h; y1 = gy - h; x2 = gx + h; y2 = gy + h
            gsx1[sl] = x1; gsy1[sl] = y1; gsx2[sl] = x2; gsy2[sl] = y2
            gsA[sl] = (x2 - x1) * (y2 - y1)
            rad = gt * D2R
            cs = _cos(rad); sn = _sin(rad)
            ca = jnp.abs(cs); sa = jnp.abs(sn)
            ew = (gw * ca + gh * sa) * np.float32(0.5)
            eh = (gw * sa + gh * ca) * np.float32(0.5)
            bx1 = gx - ew; by1 = gy - eh; bx2 = gx + ew; by2 = gy + eh
            gax1[sl] = bx1; gay1[sl] = by1; gax2[sl] = bx2; gay2[sl] = by2
            gaA[sl] = (bx2 - bx1) * (by2 - by1)
            ggx[sl] = gx; ggy[sl] = gy
            glw[sl] = _log(jnp.maximum(gw, 1.0))
            glh[sl] = _log(jnp.maximum(gh, 1.0))
            gtn[sl] = sn / cs
        nvalid_s = jnp.sum(nvalid)

        def pre_a(c, _):
            rows = c * 16 + iota
            sl = pl.ds(c * 16, 16)
            ax = plsc.load_gather(a_aos, [rows * 5])
            ay = plsc.load_gather(a_aos, [rows * 5 + 1])
            aw = plsc.load_gather(a_aos, [rows * 5 + 2])
            ah = plsc.load_gather(a_aos, [rows * 5 + 3])
            at = plsc.load_gather(a_aos, [rows * 5 + 4])
            ss = jnp.maximum(aw, ah)
            h = ss * np.float32(0.5)
            x1 = ax - h; y1 = ay - h; x2 = ax + h; y2 = ay + h
            asx1[sl] = x1; asy1[sl] = y1; asx2[sl] = x2; asy2[sl] = y2
            asA[sl] = (x2 - x1) * (y2 - y1)
            rad = at * D2R
            cs = _cos(rad); sn = _sin(rad)
            ca = jnp.abs(cs); sa = jnp.abs(sn)
            ew = (aw * ca + ah * sa) * np.float32(0.5)
            eh = (aw * sa + ah * ca) * np.float32(0.5)
            bx1 = ax - ew; by1 = ay - eh; bx2 = ax + ew; by2 = ay + eh
            aax1[sl] = bx1; aay1[sl] = by1; aax2[sl] = bx2; aay2[sl] = by2
            aaA[sl] = (bx2 - bx1) * (by2 - by1)
            ew2 = jnp.maximum(aw, 1.0); eh2 = jnp.maximum(ah, 1.0)
            aew[sl] = ew2; aeh[sl] = eh2
            alw[sl] = _log(ew2); alh[sl] = _log(eh2)
            atn[sl] = sn / cs
            amax[sl] = jnp.full((16,), FMIN, F32)
            aarg[sl] = jnp.zeros((16,), I32)
            extra[sl] = zf
            return 0
        lax.fori_loop(0, CH, pre_a, 0)

        def m_body(m, _):
            mi = jnp.full((16,), m, I32)
            bsx1 = plsc.load_gather(gsx1, [mi])
            bsy1 = plsc.load_gather(gsy1, [mi])
            bsx2 = plsc.load_gather(gsx2, [mi])
            bsy2 = plsc.load_gather(gsy2, [mi])
            bsA = plsc.load_gather(gsA, [mi])
            bax1 = plsc.load_gather(gax1, [mi])
            bay1 = plsc.load_gather(gay1, [mi])
            bax2 = plsc.load_gather(gax2, [mi])
            bay2 = plsc.load_gather(gay2, [mi])
            baA = plsc.load_gather(gaA, [mi])
            bval = plsc.load_gather(gvalid, [mi]) > np.float32(0.5)

            def c_body(c, carry):
                gmx, grc = carry
                sl = pl.ds(c * 16, 16)
                ix1 = jnp.maximum(asx1[sl], bsx1)
                iy1 = jnp.maximum(asy1[sl], bsy1)
                ix2 = jnp.minimum(asx2[sl], bsx2)
                iy2 = jnp.minimum(asy2[sl], bsy2)
                iw = jnp.maximum(ix2 - ix1, 0.0)
                ih = jnp.maximum(iy2 - iy1, 0.0)
                inter = iw * ih
                union = asA[sl] + bsA - inter
                ind_ok = inter >= np.float32(0.1) * union
                jx1 = jnp.maximum(aax1[sl], bax1)
                jy1 = jnp.maximum(aay1[sl], bay1)
                jx2 = jnp.minimum(aax2[sl], bax2)
                jy2 = jnp.minimum(aay2[sl], bay2)
                jw = jnp.maximum(jx2 - jx1, 0.0)
                jh = jnp.maximum(jy2 - jy1, 0.0)
                inter2 = jw * jh
                union2 = aaA[sl] + baA - inter2
                iou = inter2 / union2
                ovv = jnp.where(ind_ok, iou, 0.0)
                ovv = jnp.where(bval, ovv, jnp.full((16,), NEG, F32))
                am = amax[sl]
                take = ovv > am
                amax[sl] = jnp.where(take, ovv, am)
                aarg[sl] = jnp.where(take, mi, aarg[sl])
                t2 = ovv > gmx
                gmx = jnp.where(t2, ovv, gmx)
                grc = jnp.where(t2, jnp.full((16,), c, I32), grc)
                return gmx, grc

            gmx, grc = lax.fori_loop(
                0, CH, c_body,
                (jnp.full((16,), FMIN, F32), jnp.zeros((16,), I32)))
            topv = jnp.max(gmx)
            cand = jnp.where(gmx == topv, grc * 16 + iota + base,
                             jnp.full((16,), 2147483647, I32))
            argi = jnp.min(cand)
            lane0 = iota == 0
            plsc.store_scatter(wgmax, [mi], jnp.full((16,), topv, F32), mask=lane0)
            plsc.store_scatter(wgarg, [mi], jnp.full((16,), argi, I32), mask=lane0)
            return 0
        lax.fori_loop(0, M, m_body, 0)

        pltpu.sync_copy(wgmax, sh_gmax.at[pl.ds(s * M, M)])
        pltpu.sync_copy(wgarg, sh_garg.at[pl.ds(s * M, M)])
        plsc.subcore_barrier()
        pltpu.sync_copy(sh_gmax, cgmax)
        pltpu.sync_copy(sh_garg, cgarg)
        ones = jnp.ones((16,), F32)
        for g in range(M // 16):
            bm = cgmax[pl.ds(g * 16, 16)]
            ba = cgarg[pl.ds(g * 16, 16)]
            for w in range(1, NW):
                wm = cgmax[pl.ds(w * M + g * 16, 16)]
                wa = cgarg[pl.ds(w * M + g * 16, 16)]
                t = wm > bm
                ba = jnp.where(t, wa, ba)
                bm = jnp.where(t, wm, bm)
            gv = gvalid[pl.ds(g * 16, 16)] > np.float32(0.5)
            force = (bm < np.float32(0.5)) & gv
            idl = ba - base
            msk = force & (idl >= 0) & (idl < PW)
            idl = jnp.clip(idl, 0, PW - 1)
            plsc.store_scatter(extra, [idl], ones, mask=msk)

        def p2_body(c, carry):
            accL, accP = carry
            sl = pl.ds(c * 16, 16)
            rows = c * 16 + iota
            am = amax[sl]
            ag = aarg[sl]
            posb = (am >= np.float32(0.5)) | (extra[sl] > np.float32(0.5))
            gxv = plsc.load_gather(ggx, [ag])
            gyv = plsc.load_gather(ggy, [ag])
            glwv = plsc.load_gather(glw, [ag])
            glhv = plsc.load_gather(glh, [ag])
            gtnv = plsc.load_gather(gtn, [ag])
            axv = plsc.load_gather(a_aos, [rows * 5])
            ayv = plsc.load_gather(a_aos, [rows * 5 + 1])
            tdx = np.float32(10.0) * (gxv - axv) / aew[sl]
            tdy = np.float32(10.0) * (gyv - ayv) / aeh[sl]
            tdw = np.float32(10.0) * (glwv - alw[sl])
            tdh = np.float32(10.0) * (glhv - alh[sl])
            tdt = np.float32(15.0) * (gtnv - atn[sl])
            ssum = zf
            for k, td in enumerate((tdx, tdy, tdw, tdh, tdt)):
                rv = plsc.load_gather(r_aos, [rows * 5 + k])
                d = jnp.abs(rv - td)
                e = jnp.where(d < BETA, HOB * d * d, d - HB)
                ssum = ssum + e
            accL = accL + jnp.where(posb, ssum, zf)
            accP = accP + jnp.where(posb, jnp.ones((16,), F32), zf)
            return accL, accP
        accL, accP = lax.fori_loop(0, CH, p2_body, (zf, zf))

        lsum = jnp.sum(accL)
        psum = jnp.sum(accP)
        row = jnp.where(iota == 0, jnp.full((16,), lsum, F32),
                        jnp.where(iota == 1, jnp.full((16,), psum, F32), zf))
        st16[pl.ds(0, 16)] = row
        pltpu.sync_copy(st16, sh_acc.at[pl.ds(s * 16, 16)])
        plsc.subcore_barrier()

        @pl.when(s == 0)
        def _():
            pltpu.sync_copy(sh_acc, cacc)
            tot = zf
            for w in range(NW):
                tot = tot + cacc[pl.ds(w * 16, 16)]
            st16[pl.ds(0, 16)] = tot
            lv = plsc.load_gather(st16, [jnp.zeros((16,), I32)])
            pv = plsc.load_gather(st16, [jnp.ones((16,), I32)])
            denom = jnp.maximum(pv * np.float32(5.0), 1.0)
            res = lv / denom
            ok = (pv > 0.0) & (jnp.full((16,), nvalid_s, F32) > 0.0)
            outrow = jnp.where(ok & (iota == 0), res, zf)
            st16[pl.ds(0, 16)] = outrow
            pltpu.sync_copy(st16, out_hbm.at[j])

    return body


_body = _make_body()


@jax.jit
def kernel(regressions, anchors, annotations):
    pad_a = jnp.zeros((B, NP - N, 5), F32)
    pad_a = pad_a.at[:, :, 0].set(-1e6).at[:, :, 1].set(-1e6)
    pad_a = pad_a.at[:, :, 2].set(16.0).at[:, :, 3].set(16.0)
    anc = jnp.concatenate([anchors, pad_a], axis=1).reshape(-1)
    reg = jnp.concatenate(
        [regressions, jnp.zeros((B, NP - N, 5), F32)], axis=1).reshape(-1)
    ann = annotations.reshape(-1)
    out = _body(reg, anc, ann)
    return jnp.mean(out[:, 0], keepdims=True)


# speedup vs baseline: 2.6738x; 1.1214x over previous
import functools

import numpy as np
import jax
import jax.numpy as jnp
from jax import lax
from jax.experimental import pallas as pl
from jax.experimental.pallas import tpu as pltpu
from jax.experimental.pallas import tpu_sc as plsc

F32 = jnp.float32
I32 = jnp.int32

B = 2
N = 20000
NP = 20480
M = 64
NW = 16
PW = NP // NW
CH = PW // 16

D2R = np.float32(np.pi / 180.0)
BETA = np.float32(1.0 / 9.0)
HOB = np.float32(0.5 / np.float32(1.0 / 9.0))
HB = np.float32(0.5 * np.float32(1.0 / 9.0))
LN2 = np.float32(0.6931471805599453)
NEG = np.float32(-1e9)
FMIN = np.float32(-3.4e38)


def _sin(x):
    x2 = x * x
    return x * (1.0 + x2 * (np.float32(-1 / 6) + x2 * (np.float32(1 / 120)
            + x2 * (np.float32(-1 / 5040) + x2 * np.float32(1 / 362880)))))


def _cos(x):
    x2 = x * x
    return 1.0 + x2 * (np.float32(-0.5) + x2 * (np.float32(1 / 24)
            + x2 * (np.float32(-1 / 720) + x2 * np.float32(1 / 40320))))


def _log(w):
    xi = plsc.bitcast(w, I32)
    e = lax.shift_right_logical(xi, 23) - 127
    m = plsc.bitcast(jnp.bitwise_or(jnp.bitwise_and(xi, 0x7FFFFF), 0x3F800000), F32)
    big = m > np.float32(1.4142135)
    m = jnp.where(big, m * np.float32(0.5), m)
    e = e + big.astype(I32)
    z = (m - 1.0) / (m + 1.0)
    z2 = z * z
    lm = 2.0 * z * (1.0 + z2 * (np.float32(1 / 3) + z2 * (np.float32(1 / 5)
            + z2 * (np.float32(1 / 7) + z2 * np.float32(1 / 9)))))
    return e.astype(F32) * LN2 + lm


def _make_body():
    mesh = plsc.VectorSubcoreMesh(core_axis_name="c", subcore_axis_name="s")
    scratch = [
        pltpu.VMEM((PW * 5,), F32),
        pltpu.VMEM((PW * 5,), F32),
        pltpu.VMEM((M * 6,), F32),
        pltpu.VMEM((PW,), F32),
        pltpu.VMEM((PW,), F32),
        pltpu.VMEM((PW,), F32),
        pltpu.VMEM((PW,), F32),
        pltpu.VMEM((PW,), F32),
        pltpu.VMEM((PW,), F32),
        pltpu.VMEM((PW,), F32),
        pltpu.VMEM((PW,), F32),
        pltpu.VMEM((PW,), F32),
        pltpu.VMEM((PW,), F32),
        pltpu.VMEM((PW,), F32),
        pltpu.VMEM((PW,), F32),
        pltpu.VMEM((PW,), F32),
        pltpu.VMEM((PW,), F32),
        pltpu.VMEM((PW,), F32),
        pltpu.VMEM((PW,), F32),
        pltpu.VMEM((PW,), I32),
        pltpu.VMEM((PW,), F32),
        pltpu.VMEM((M,), F32),
        pltpu.VMEM((M,), F32),
        pltpu.VMEM((M,), F32),
        pltpu.VMEM((M,), F32),
        pltpu.VMEM((M,), F32),
        pltpu.VMEM((M,), F32),
        pltpu.VMEM((M,), F32),
        pltpu.VMEM((M,), F32),
        pltpu.VMEM((M,), F32),
        pltpu.VMEM((M,), F32),
        pltpu.VMEM((M,), F32),
        pltpu.VMEM((M,), F32),
        pltpu.VMEM((M,), F32),
        pltpu.VMEM((M,), F32),
        pltpu.VMEM((M,), F32),
        pltpu.VMEM((M,), F32),
        pltpu.VMEM((M,), F32),
        pltpu.VMEM((M,), I32),
        pltpu.VMEM((NW * M,), F32),
        pltpu.VMEM((NW * M,), I32),
        pltpu.VMEM((NW * 16,), F32),
        pltpu.VMEM((16,), F32),
        pltpu.VMEM_SHARED((NW * M,), F32),
        pltpu.VMEM_SHARED((NW * M,), I32),
        pltpu.VMEM_SHARED((NW * 16,), F32),
    ]

    @functools.partial(
        pl.kernel, mesh=mesh,
        out_type=jax.ShapeDtypeStruct((B, 16), F32),
        scratch_types=scratch,
        compiler_params=pltpu.CompilerParams(needs_layout_passes=False),
    )
    def body(reg_hbm, anc_hbm, ann_hbm, out_hbm,
             a_aos, r_aos, ann_aos,
             asx1, asy1, asx2, asy2, asA, aax1, aay1, aax2, aay2, aaA,
             aew, aeh, alw, alh, atn,
             amax, aarg, extra,
             gsx1, gsy1, gsx2, gsy2, gsA, gax1, gay1, gax2, gay2, gaA,
             gvalid, ggx, ggy, glw, glh, gtn,
             wgmax, wgarg, cgmax, cgarg, cacc, st16,
             sh_gmax, sh_garg, sh_acc):
        j = lax.axis_index("c")
        s = lax.axis_index("s")
        iota = lax.iota(I32, 16)
        zf = jnp.zeros((16,), F32)
        base = s * PW

        REAL_LAST = N - 15 * PW
        @pl.when(s < NW - 1)
        def _():
            pltpu.sync_copy(anc_hbm.at[pl.ds(j * (N * 5) + base * 5, PW * 5)], a_aos)
            pltpu.sync_copy(reg_hbm.at[pl.ds(j * (N * 5) + base * 5, PW * 5)], r_aos)
        @pl.when(s == NW - 1)
        def _():
            pltpu.sync_copy(anc_hbm.at[pl.ds(j * (N * 5) + base * 5, REAL_LAST * 5)],
                            a_aos.at[pl.ds(0, REAL_LAST * 5)])
            pltpu.sync_copy(reg_hbm.at[pl.ds(j * (N * 5) + base * 5, REAL_LAST * 5)],
                            r_aos.at[pl.ds(0, REAL_LAST * 5)])
        pltpu.sync_copy(ann_hbm.at[pl.ds(j * (M * 6), M * 6)], ann_aos)

        nvalid = zf
        for g in range(M // 16):
            rows = g * 16 + iota
            gx = plsc.load_gather(ann_aos, [rows * 6])
            gy = plsc.load_gather(ann_aos, [rows * 6 + 1])
            gw = plsc.load_gather(ann_aos, [rows * 6 + 2])
            gh = plsc.load_gather(ann_aos, [rows * 6 + 3])
            gt = plsc.load_gather(ann_aos, [rows * 6 + 4])
            gl = plsc.load_gather(ann_aos, [rows * 6 + 5])
            sl = pl.ds(g * 16, 16)
            vf = jnp.where(gl != np.float32(-1.0), jnp.ones((16,), F32), zf)
            gvalid[sl] = vf
            nvalid = nvalid + vf
            gs = jnp.maximum(gw, gh)
            h = gs * np.float32(0.5)
            x1 = gx - h; y1 = gy - h; x2 = gx + h; y2 = gy + h
            gsx1[sl] = x1; gsy1[sl] = y1; gsx2[sl] = x2; gsy2[sl] = y2
            gsA[sl] = (x2 - x1) * (y2 - y1)
            rad = gt * D2R
            cs = _cos(rad); sn = _sin(rad)
            ca = jnp.abs(cs); sa = jnp.abs(sn)
            ew = (gw * ca + gh * sa) * np.float32(0.5)
            eh = (gw * sa + gh * ca) * np.float32(0.5)
            bx1 = gx - ew; by1 = gy - eh; bx2 = gx + ew; by2 = gy + eh
            gax1[sl] = bx1; gay1[sl] = by1; gax2[sl] = bx2; gay2[sl] = by2
            gaA[sl] = (bx2 - bx1) * (by2 - by1)
            ggx[sl] = gx; ggy[sl] = gy
            glw[sl] = _log(jnp.maximum(gw, 1.0))
            glh[sl] = _log(jnp.maximum(gh, 1.0))
            gtn[sl] = sn / cs
        nvalid_s = jnp.sum(nvalid)

        def pre_a(c, _):
            rows = c * 16 + iota
            sl = pl.ds(c * 16, 16)
            ax = plsc.load_gather(a_aos, [rows * 5])
            ay = plsc.load_gather(a_aos, [rows * 5 + 1])
            aw = plsc.load_gather(a_aos, [rows * 5 + 2])
            ah = plsc.load_gather(a_aos, [rows * 5 + 3])
            at = plsc.load_gather(a_aos, [rows * 5 + 4])
            padm = (rows + base) >= N
            ax = jnp.where(padm, jnp.full((16,), np.float32(-1e6), F32), ax)
            ay = jnp.where(padm, jnp.full((16,), np.float32(-1e6), F32), ay)
            aw = jnp.where(padm, jnp.full((16,), np.float32(16.0), F32), aw)
            ah = jnp.where(padm, jnp.full((16,), np.float32(16.0), F32), ah)
            at = jnp.where(padm, zf, at)
            ss = jnp.maximum(aw, ah)
            h = ss * np.float32(0.5)
            x1 = ax - h; y1 = ay - h; x2 = ax + h; y2 = ay + h
            asx1[sl] = x1; asy1[sl] = y1; asx2[sl] = x2; asy2[sl] = y2
            asA[sl] = (x2 - x1) * (y2 - y1)
            rad = at * D2R
            cs = _cos(rad); sn = _sin(rad)
            ca = jnp.abs(cs); sa = jnp.abs(sn)
            ew = (aw * ca + ah * sa) * np.float32(0.5)
            eh = (aw * sa + ah * ca) * np.float32(0.5)
            bx1 = ax - ew; by1 = ay - eh; bx2 = ax + ew; by2 = ay + eh
            aax1[sl] = bx1; aay1[sl] = by1; aax2[sl] = bx2; aay2[sl] = by2
            aaA[sl] = (bx2 - bx1) * (by2 - by1)
            ew2 = jnp.maximum(aw, 1.0); eh2 = jnp.maximum(ah, 1.0)
            aew[sl] = ew2; aeh[sl] = eh2
            alw[sl] = _log(ew2); alh[sl] = _log(eh2)
            atn[sl] = sn / cs
            amax[sl] = jnp.full((16,), FMIN, F32)
            aarg[sl] = jnp.zeros((16,), I32)
            extra[sl] = zf
            return 0
        lax.fori_loop(0, CH, pre_a, 0)

        def m_body(m, _):
            mi = jnp.full((16,), m, I32)
            bsx1 = plsc.load_gather(gsx1, [mi])
            bsy1 = plsc.load_gather(gsy1, [mi])
            bsx2 = plsc.load_gather(gsx2, [mi])
            bsy2 = plsc.load_gather(gsy2, [mi])
            bsA = plsc.load_gather(gsA, [mi])
            bax1 = plsc.load_gather(gax1, [mi])
            bay1 = plsc.load_gather(gay1, [mi])
            bax2 = plsc.load_gather(gax2, [mi])
            bay2 = plsc.load_gather(gay2, [mi])
            baA = plsc.load_gather(gaA, [mi])
            bval = plsc.load_gather(gvalid, [mi]) > np.float32(0.5)

            def c_body(c, carry):
                gmx, grc = carry
                sl = pl.ds(c * 16, 16)
                ix1 = jnp.maximum(asx1[sl], bsx1)
                iy1 = jnp.maximum(asy1[sl], bsy1)
                ix2 = jnp.minimum(asx2[sl], bsx2)
                iy2 = jnp.minimum(asy2[sl], bsy2)
                iw = jnp.maximum(ix2 - ix1, 0.0)
                ih = jnp.maximum(iy2 - iy1, 0.0)
                inter = iw * ih
                union = asA[sl] + bsA - inter
                ind_ok = inter >= np.float32(0.1) * union
                jx1 = jnp.maximum(aax1[sl], bax1)
                jy1 = jnp.maximum(aay1[sl], bay1)
                jx2 = jnp.minimum(aax2[sl], bax2)
                jy2 = jnp.minimum(aay2[sl], bay2)
                jw = jnp.maximum(jx2 - jx1, 0.0)
                jh = jnp.maximum(jy2 - jy1, 0.0)
                inter2 = jw * jh
                union2 = aaA[sl] + baA - inter2
                iou = inter2 / union2
                ovv = jnp.where(ind_ok, iou, 0.0)
                ovv = jnp.where(bval, ovv, jnp.full((16,), NEG, F32))
                am = amax[sl]
                take = ovv > am
                amax[sl] = jnp.where(take, ovv, am)
                aarg[sl] = jnp.where(take, mi, aarg[sl])
                t2 = ovv > gmx
                gmx = jnp.where(t2, ovv, gmx)
                grc = jnp.where(t2, jnp.full((16,), c, I32), grc)
                return gmx, grc

            gmx, grc = lax.fori_loop(
                0, CH, c_body,
                (jnp.full((16,), FMIN, F32), jnp.zeros((16,), I32)))
            topv = jnp.max(gmx)
            cand = jnp.where(gmx == topv, grc * 16 + iota + base,
                             jnp.full((16,), 2147483647, I32))
            argi = jnp.min(cand)
            lane0 = iota == 0
            plsc.store_scatter(wgmax, [mi], jnp.full((16,), topv, F32), mask=lane0)
            plsc.store_scatter(wgarg, [mi], jnp.full((16,), argi, I32), mask=lane0)
            return 0
        lax.fori_loop(0, M, m_body, 0)

        pltpu.sync_copy(wgmax, sh_gmax.at[pl.ds(s * M, M)])
        pltpu.sync_copy(wgarg, sh_garg.at[pl.ds(s * M, M)])
        plsc.subcore_barrier()
        pltpu.sync_copy(sh_gmax, cgmax)
        pltpu.sync_copy(sh_garg, cgarg)
        ones = jnp.ones((16,), F32)
        for g in range(M // 16):
            bm = cgmax[pl.ds(g * 16, 16)]
            ba = cgarg[pl.ds(g * 16, 16)]
            for w in range(1, NW):
                wm = cgmax[pl.ds(w * M + g * 16, 16)]
                wa = cgarg[pl.ds(w * M + g * 16, 16)]
                t = wm > bm
                ba = jnp.where(t, wa, ba)
                bm = jnp.where(t, wm, bm)
            gv = gvalid[pl.ds(g * 16, 16)] > np.float32(0.5)
            force = (bm < np.float32(0.5)) & gv
            idl = ba - base
            msk = force & (idl >= 0) & (idl < PW)
            idl = jnp.clip(idl, 0, PW - 1)
            plsc.store_scatter(extra, [idl], ones, mask=msk)

        def p2_body(c, carry):
            accL, accP = carry
            sl = pl.ds(c * 16, 16)
            rows = c * 16 + iota
            am = amax[sl]
            ag = aarg[sl]
            posb = (am >= np.float32(0.5)) | (extra[sl] > np.float32(0.5))
            gxv = plsc.load_gather(ggx, [ag])
            gyv = plsc.load_gather(ggy, [ag])
            glwv = plsc.load_gather(glw, [ag])
            glhv = plsc.load_gather(glh, [ag])
            gtnv = plsc.load_gather(gtn, [ag])
            axv = plsc.load_gather(a_aos, [rows * 5])
            ayv = plsc.load_gather(a_aos, [rows * 5 + 1])
            tdx = np.float32(10.0) * (gxv - axv) / aew[sl]
            tdy = np.float32(10.0) * (gyv - ayv) / aeh[sl]
            tdw = np.float32(10.0) * (glwv - alw[sl])
            tdh = np.float32(10.0) * (glhv - alh[sl])
            tdt = np.float32(15.0) * (gtnv - atn[sl])
            ssum = zf
            for k, td in enumerate((tdx, tdy, tdw, tdh, tdt)):
                rv = plsc.load_gather(r_aos, [rows * 5 + k])
                d = jnp.abs(rv - td)
                e = jnp.where(d < BETA, HOB * d * d, d - HB)
                ssum = ssum + e
            accL = accL + jnp.where(posb, ssum, zf)
            accP = accP + jnp.where(posb, jnp.ones((16,), F32), zf)
            return accL, accP
        accL, accP = lax.fori_loop(0, CH, p2_body, (zf, zf))

        lsum = jnp.sum(accL)
        psum = jnp.sum(accP)
        row = jnp.where(iota == 0, jnp.full((16,), lsum, F32),
                        jnp.where(iota == 1, jnp.full((16,), psum, F32), zf))
        st16[pl.ds(0, 16)] = row
        pltpu.sync_copy(st16, sh_acc.at[pl.ds(s * 16, 16)])
        plsc.subcore_barrier()

        @pl.when(s == 0)
        def _():
            pltpu.sync_copy(sh_acc, cacc)
            tot = zf
            for w in range(NW):
                tot = tot + cacc[pl.ds(w * 16, 16)]
            st16[pl.ds(0, 16)] = tot
            lv = plsc.load_gather(st16, [jnp.zeros((16,), I32)])
            pv = plsc.load_gather(st16, [jnp.ones((16,), I32)])
            denom = jnp.maximum(pv * np.float32(5.0), 1.0)
            res = lv / denom
            ok = (pv > 0.0) & (jnp.full((16,), nvalid_s, F32) > 0.0)
            outrow = jnp.where(ok & (iota == 0), res, zf)
            st16[pl.ds(0, 16)] = outrow
            pltpu.sync_copy(st16, out_hbm.at[j])

    return body


_body = _make_body()


@jax.jit
def kernel(regressions, anchors, annotations):
    reg = regressions.reshape(-1)
    anc = anchors.reshape(-1)
    ann = annotations.reshape(-1)
    out = _body(reg, anc, ann)
    return jnp.mean(out[:, 0], keepdims=True)


# speedup vs baseline: 4.4657x; 1.6702x over previous
import functools

import numpy as np
import jax
import jax.numpy as jnp
from jax import lax
from jax.experimental import pallas as pl
from jax.experimental.pallas import tpu as pltpu
from jax.experimental.pallas import tpu_sc as plsc

F32 = jnp.float32
I32 = jnp.int32

B = 2
N = 20000
NP = 20480
M = 64
NW = 16
PW = NP // NW
CH = PW // 16

D2R = np.float32(np.pi / 180.0)
BETA = np.float32(1.0 / 9.0)
HOB = np.float32(0.5 / np.float32(1.0 / 9.0))
HB = np.float32(0.5 * np.float32(1.0 / 9.0))
LN2 = np.float32(0.6931471805599453)
NEG = np.float32(-1e9)
FMIN = np.float32(-3.4e38)


def _sin(x):
    x2 = x * x
    return x * (1.0 + x2 * (np.float32(-1 / 6) + x2 * (np.float32(1 / 120)
            + x2 * (np.float32(-1 / 5040) + x2 * np.float32(1 / 362880)))))


def _cos(x):
    x2 = x * x
    return 1.0 + x2 * (np.float32(-0.5) + x2 * (np.float32(1 / 24)
            + x2 * (np.float32(-1 / 720) + x2 * np.float32(1 / 40320))))


def _log(w):
    xi = plsc.bitcast(w, I32)
    e = lax.shift_right_logical(xi, 23) - 127
    m = plsc.bitcast(jnp.bitwise_or(jnp.bitwise_and(xi, 0x7FFFFF), 0x3F800000), F32)
    big = m > np.float32(1.4142135)
    m = jnp.where(big, m * np.float32(0.5), m)
    e = e + big.astype(I32)
    z = (m - 1.0) / (m + 1.0)
    z2 = z * z
    lm = 2.0 * z * (1.0 + z2 * (np.float32(1 / 3) + z2 * (np.float32(1 / 5)
            + z2 * (np.float32(1 / 7) + z2 * np.float32(1 / 9)))))
    return e.astype(F32) * LN2 + lm


def _make_body():
    mesh = plsc.VectorSubcoreMesh(core_axis_name="c", subcore_axis_name="s")
    scratch = [
        pltpu.VMEM((PW * 5,), F32),
        pltpu.VMEM((PW * 5,), F32),
        pltpu.VMEM((M * 6,), F32),
        pltpu.VMEM((PW,), F32),
        pltpu.VMEM((PW,), F32),
        pltpu.VMEM((PW,), F32),
        pltpu.VMEM((PW,), F32),
        pltpu.VMEM((PW,), F32),
        pltpu.VMEM((PW,), F32),
        pltpu.VMEM((PW,), F32),
        pltpu.VMEM((PW,), F32),
        pltpu.VMEM((PW,), F32),
        pltpu.VMEM((PW,), F32),
        pltpu.VMEM((PW,), F32),
        pltpu.VMEM((PW,), F32),
        pltpu.VMEM((PW,), F32),
        pltpu.VMEM((PW,), F32),
        pltpu.VMEM((PW,), F32),
        pltpu.VMEM((PW,), F32),
        pltpu.VMEM((PW,), I32),
        pltpu.VMEM((PW,), F32),
        pltpu.VMEM((M,), F32),
        pltpu.VMEM((M,), F32),
        pltpu.VMEM((M,), F32),
        pltpu.VMEM((M,), F32),
        pltpu.VMEM((M,), F32),
        pltpu.VMEM((M,), F32),
        pltpu.VMEM((M,), F32),
        pltpu.VMEM((M,), F32),
        pltpu.VMEM((M,), F32),
        pltpu.VMEM((M,), F32),
        pltpu.VMEM((M,), F32),
        pltpu.VMEM((M,), F32),
        pltpu.VMEM((M,), F32),
        pltpu.VMEM((M,), F32),
        pltpu.VMEM((M,), F32),
        pltpu.VMEM((M,), F32),
        pltpu.VMEM((M,), F32),
        pltpu.VMEM((M,), I32),
        pltpu.VMEM((NW * M,), F32),
        pltpu.VMEM((NW * M,), I32),
        pltpu.VMEM((NW * 16,), F32),
        pltpu.VMEM((16,), F32),
        pltpu.VMEM_SHARED((NW * M,), F32),
        pltpu.VMEM_SHARED((NW * M,), I32),
        pltpu.VMEM_SHARED((NW * 16,), F32),
    ]

    @functools.partial(
        pl.kernel, mesh=mesh,
        out_type=jax.ShapeDtypeStruct((B, 16), F32),
        scratch_types=scratch,
        compiler_params=pltpu.CompilerParams(needs_layout_passes=False),
    )
    def body(reg_hbm, anc_hbm, ann_hbm, out_hbm,
             a_aos, r_aos, ann_aos,
             asx1, asy1, asx2, asy2, asA, aax1, aay1, aax2, aay2, aaA,
             aew, aeh, alw, alh, atn,
             amax, aarg, extra,
             gsx1, gsy1, gsx2, gsy2, gsA, gax1, gay1, gax2, gay2, gaA,
             gvalid, ggx, ggy, glw, glh, gtn,
             wgmax, wgarg, cgmax, cgarg, cacc, st16,
             sh_gmax, sh_garg, sh_acc):
        j = lax.axis_index("c")
        s = lax.axis_index("s")
        iota = lax.iota(I32, 16)
        zf = jnp.zeros((16,), F32)
        base = s * PW

        REAL_LAST = N - 15 * PW
        @pl.when(s < NW - 1)
        def _():
            for k in range(5):
                pltpu.sync_copy(anc_hbm.at[pl.ds(j * (N * 5) + k * N + base, PW)],
                                a_aos.at[pl.ds(k * PW, PW)])
                pltpu.sync_copy(reg_hbm.at[pl.ds(j * (N * 5) + k * N + base, PW)],
                                r_aos.at[pl.ds(k * PW, PW)])
        @pl.when(s == NW - 1)
        def _():
            for k in range(5):
                pltpu.sync_copy(anc_hbm.at[pl.ds(j * (N * 5) + k * N + base, REAL_LAST)],
                                a_aos.at[pl.ds(k * PW, REAL_LAST)])
                pltpu.sync_copy(reg_hbm.at[pl.ds(j * (N * 5) + k * N + base, REAL_LAST)],
                                r_aos.at[pl.ds(k * PW, REAL_LAST)])
        pltpu.sync_copy(ann_hbm.at[pl.ds(j * (M * 6), M * 6)], ann_aos)

        nvalid = zf
        for g in range(M // 16):
            gx = ann_aos[pl.ds(0 * M + g * 16, 16)]
            gy = ann_aos[pl.ds(1 * M + g * 16, 16)]
            gw = ann_aos[pl.ds(2 * M + g * 16, 16)]
            gh = ann_aos[pl.ds(3 * M + g * 16, 16)]
            gt = ann_aos[pl.ds(4 * M + g * 16, 16)]
            gl = ann_aos[pl.ds(5 * M + g * 16, 16)]
            sl = pl.ds(g * 16, 16)
            vf = jnp.where(gl != np.float32(-1.0), jnp.ones((16,), F32), zf)
            gvalid[sl] = vf
            nvalid = nvalid + vf
            gs = jnp.maximum(gw, gh)
            h = gs * np.float32(0.5)
            x1 = gx - h; y1 = gy - h; x2 = gx + h; y2 = gy + h
            gsx1[sl] = x1; gsy1[sl] = y1; gsx2[sl] = x2; gsy2[sl] = y2
            gsA[sl] = (x2 - x1) * (y2 - y1)
            rad = gt * D2R
            cs = _cos(rad); sn = _sin(rad)
            ca = jnp.abs(cs); sa = jnp.abs(sn)
            ew = (gw * ca + gh * sa) * np.float32(0.5)
            eh = (gw * sa + gh * ca) * np.float32(0.5)
            bx1 = gx - ew; by1 = gy - eh; bx2 = gx + ew; by2 = gy + eh
            gax1[sl] = bx1; gay1[sl] = by1; gax2[sl] = bx2; gay2[sl] = by2
            gaA[sl] = (bx2 - bx1) * (by2 - by1)
            ggx[sl] = gx; ggy[sl] = gy
            glw[sl] = _log(jnp.maximum(gw, 1.0))
            glh[sl] = _log(jnp.maximum(gh, 1.0))
            gtn[sl] = sn / cs
        nvalid_s = jnp.sum(nvalid)

        def pre_a(c, _):
            rows = c * 16 + iota
            sl = pl.ds(c * 16, 16)
            ax = a_aos[pl.ds(0 * PW + c * 16, 16)]
            ay = a_aos[pl.ds(1 * PW + c * 16, 16)]
            aw = a_aos[pl.ds(2 * PW + c * 16, 16)]
            ah = a_aos[pl.ds(3 * PW + c * 16, 16)]
            at = a_aos[pl.ds(4 * PW + c * 16, 16)]
            padm = (rows + base) >= N
            ax = jnp.where(padm, jnp.full((16,), np.float32(-1e6), F32), ax)
            ay = jnp.where(padm, jnp.full((16,), np.float32(-1e6), F32), ay)
            aw = jnp.where(padm, jnp.full((16,), np.float32(16.0), F32), aw)
            ah = jnp.where(padm, jnp.full((16,), np.float32(16.0), F32), ah)
            at = jnp.where(padm, zf, at)
            ss = jnp.maximum(aw, ah)
            h = ss * np.float32(0.5)
            x1 = ax - h; y1 = ay - h; x2 = ax + h; y2 = ay + h
            asx1[sl] = x1; asy1[sl] = y1; asx2[sl] = x2; asy2[sl] = y2
            asA[sl] = (x2 - x1) * (y2 - y1)
            rad = at * D2R
            cs = _cos(rad); sn = _sin(rad)
            ca = jnp.abs(cs); sa = jnp.abs(sn)
            ew = (aw * ca + ah * sa) * np.float32(0.5)
            eh = (aw * sa + ah * ca) * np.float32(0.5)
            bx1 = ax - ew; by1 = ay - eh; bx2 = ax + ew; by2 = ay + eh
            aax1[sl] = bx1; aay1[sl] = by1; aax2[sl] = bx2; aay2[sl] = by2
            aaA[sl] = (bx2 - bx1) * (by2 - by1)
            ew2 = jnp.maximum(aw, 1.0); eh2 = jnp.maximum(ah, 1.0)
            aew[sl] = ew2; aeh[sl] = eh2
            alw[sl] = _log(ew2); alh[sl] = _log(eh2)
            atn[sl] = sn / cs
            amax[sl] = jnp.full((16,), FMIN, F32)
            aarg[sl] = jnp.zeros((16,), I32)
            extra[sl] = zf
            return 0
        lax.fori_loop(0, CH, pre_a, 0)

        def m_body(m, _):
            mi = jnp.full((16,), m, I32)
            bsx1 = plsc.load_gather(gsx1, [mi])
            bsy1 = plsc.load_gather(gsy1, [mi])
            bsx2 = plsc.load_gather(gsx2, [mi])
            bsy2 = plsc.load_gather(gsy2, [mi])
            bsA = plsc.load_gather(gsA, [mi])
            bax1 = plsc.load_gather(gax1, [mi])
            bay1 = plsc.load_gather(gay1, [mi])
            bax2 = plsc.load_gather(gax2, [mi])
            bay2 = plsc.load_gather(gay2, [mi])
            baA = plsc.load_gather(gaA, [mi])
            bval = plsc.load_gather(gvalid, [mi]) > np.float32(0.5)

            def c_body(c, carry):
                gmx, grc = carry
                sl = pl.ds(c * 16, 16)
                ix1 = jnp.maximum(asx1[sl], bsx1)
                iy1 = jnp.maximum(asy1[sl], bsy1)
                ix2 = jnp.minimum(asx2[sl], bsx2)
                iy2 = jnp.minimum(asy2[sl], bsy2)
                iw = jnp.maximum(ix2 - ix1, 0.0)
                ih = jnp.maximum(iy2 - iy1, 0.0)
                inter = iw * ih
                union = asA[sl] + bsA - inter
                ind_ok = inter >= np.float32(0.1) * union
                jx1 = jnp.maximum(aax1[sl], bax1)
                jy1 = jnp.maximum(aay1[sl], bay1)
                jx2 = jnp.minimum(aax2[sl], bax2)
                jy2 = jnp.minimum(aay2[sl], bay2)
                jw = jnp.maximum(jx2 - jx1, 0.0)
                jh = jnp.maximum(jy2 - jy1, 0.0)
                inter2 = jw * jh
                union2 = aaA[sl] + baA - inter2
                iou = inter2 / union2
                ovv = jnp.where(ind_ok, iou, 0.0)
                ovv = jnp.where(bval, ovv, jnp.full((16,), NEG, F32))
                am = amax[sl]
                take = ovv > am
                amax[sl] = jnp.where(take, ovv, am)
                aarg[sl] = jnp.where(take, mi, aarg[sl])
                t2 = ovv > gmx
                gmx = jnp.where(t2, ovv, gmx)
                grc = jnp.where(t2, jnp.full((16,), c, I32), grc)
                return gmx, grc

            gmx, grc = lax.fori_loop(
                0, CH, c_body,
                (jnp.full((16,), FMIN, F32), jnp.zeros((16,), I32)))
            topv = jnp.max(gmx)
            cand = jnp.where(gmx == topv, grc * 16 + iota + base,
                             jnp.full((16,), 2147483647, I32))
            argi = jnp.min(cand)
            lane0 = iota == 0
            plsc.store_scatter(wgmax, [mi], jnp.full((16,), topv, F32), mask=lane0)
            plsc.store_scatter(wgarg, [mi], jnp.full((16,), argi, I32), mask=lane0)
            return 0
        lax.fori_loop(0, M, m_body, 0)

        pltpu.sync_copy(wgmax, sh_gmax.at[pl.ds(s * M, M)])
        pltpu.sync_copy(wgarg, sh_garg.at[pl.ds(s * M, M)])
        plsc.subcore_barrier()
        pltpu.sync_copy(sh_gmax, cgmax)
        pltpu.sync_copy(sh_garg, cgarg)
        ones = jnp.ones((16,), F32)
        for g in range(M // 16):
            bm = cgmax[pl.ds(g * 16, 16)]
            ba = cgarg[pl.ds(g * 16, 16)]
            for w in range(1, NW):
                wm = cgmax[pl.ds(w * M + g * 16, 16)]
                wa = cgarg[pl.ds(w * M + g * 16, 16)]
                t = wm > bm
                ba = jnp.where(t, wa, ba)
                bm = jnp.where(t, wm, bm)
            gv = gvalid[pl.ds(g * 16, 16)] > np.float32(0.5)
            force = (bm < np.float32(0.5)) & gv
            idl = ba - base
            msk = force & (idl >= 0) & (idl < PW)
            idl = jnp.clip(idl, 0, PW - 1)
            plsc.store_scatter(extra, [idl], ones, mask=msk)

        def p2_body(c, carry):
            accL, accP = carry
            sl = pl.ds(c * 16, 16)
            rows = c * 16 + iota
            am = amax[sl]
            ag = aarg[sl]
            posb = (am >= np.float32(0.5)) | (extra[sl] > np.float32(0.5))
            gxv = plsc.load_gather(ggx, [ag])
            gyv = plsc.load_gather(ggy, [ag])
            glwv = plsc.load_gather(glw, [ag])
            glhv = plsc.load_gather(glh, [ag])
            gtnv = plsc.load_gather(gtn, [ag])
            axv = a_aos[pl.ds(0 * PW + c * 16, 16)]
            ayv = a_aos[pl.ds(1 * PW + c * 16, 16)]
            tdx = np.float32(10.0) * (gxv - axv) / aew[sl]
            tdy = np.float32(10.0) * (gyv - ayv) / aeh[sl]
            tdw = np.float32(10.0) * (glwv - alw[sl])
            tdh = np.float32(10.0) * (glhv - alh[sl])
            tdt = np.float32(15.0) * (gtnv - atn[sl])
            ssum = zf
            for k, td in enumerate((tdx, tdy, tdw, tdh, tdt)):
                rv = r_aos[pl.ds(k * PW + c * 16, 16)]
                d = jnp.abs(rv - td)
                e = jnp.where(d < BETA, HOB * d * d, d - HB)
                ssum = ssum + e
            accL = accL + jnp.where(posb, ssum, zf)
            accP = accP + jnp.where(posb, jnp.ones((16,), F32), zf)
            return accL, accP
        accL, accP = lax.fori_loop(0, CH, p2_body, (zf, zf))

        lsum = jnp.sum(accL)
        psum = jnp.sum(accP)
        row = jnp.where(iota == 0, jnp.full((16,), lsum, F32),
                        jnp.where(iota == 1, jnp.full((16,), psum, F32), zf))
        st16[pl.ds(0, 16)] = row
        pltpu.sync_copy(st16, sh_acc.at[pl.ds(s * 16, 16)])
        plsc.subcore_barrier()

        @pl.when(s == 0)
        def _():
            pltpu.sync_copy(sh_acc, cacc)
            tot = zf
            for w in range(NW):
                tot = tot + cacc[pl.ds(w * 16, 16)]
            st16[pl.ds(0, 16)] = tot
            lv = plsc.load_gather(st16, [jnp.zeros((16,), I32)])
            pv = plsc.load_gather(st16, [jnp.ones((16,), I32)])
            denom = jnp.maximum(pv * np.float32(5.0), 1.0)
            res = lv / denom
            ok = (pv > 0.0) & (jnp.full((16,), nvalid_s, F32) > 0.0)
            outrow = jnp.where(ok & (iota == 0), res, zf)
            st16[pl.ds(0, 16)] = outrow
            pltpu.sync_copy(st16, out_hbm.at[j])

    return body


_body = _make_body()


@jax.jit
def kernel(regressions, anchors, annotations):
    reg = jnp.transpose(regressions, (0, 2, 1)).reshape(-1)
    anc = jnp.transpose(anchors, (0, 2, 1)).reshape(-1)
    ann = jnp.transpose(annotations, (0, 2, 1)).reshape(-1)
    out = _body(reg, anc, ann)
    return jnp.mean(out[:, 0], keepdims=True)
